# baseline (speedup 1.0000x reference)
import jax
import jax.numpy as jnp
from jax.experimental import pallas as pl
from jax.experimental.pallas import tpu as pltpu

_N = 10000
_D = 128
_BM = 400


def _proj_body(x_ref, W1_ref, proj_ref):
    proj_ref[...] = jax.lax.dot_general(
        x_ref[...], W1_ref[...],
        (((1,), (1,)), ((), ())),
        preferred_element_type=jnp.float32)


def _main_body(x_blk_ref, adj_ref, proj_ref, W2_ref, out_ref):
    nb = jax.lax.dot_general(
        adj_ref[...], proj_ref[...],
        (((1,), (0,)), ((), ())),
        preferred_element_type=jnp.float32)
    x = x_blk_ref[...]
    s = x + nb
    p = x * nb
    W2 = W2_ref[...]
    y = (jax.lax.dot_general(s, W2[:, :_D], (((1,), (1,)), ((), ())),
                             preferred_element_type=jnp.float32)
         + jax.lax.dot_general(p, W2[:, _D:], (((1,), (1,)), ((), ())),
                               preferred_element_type=jnp.float32))
    out_ref[...] = jnp.where(y >= 0.0, y, 0.01 * y)


def kernel(input, adj, W1, W2):
    proj = pl.pallas_call(
        _proj_body,
        out_shape=jax.ShapeDtypeStruct((_N, _D), jnp.float32),
    )(input, W1)
    return pl.pallas_call(
        _main_body,
        grid=(_N // _BM,),
        in_specs=[
            pl.BlockSpec((_BM, _D), lambda i: (i, 0)),
            pl.BlockSpec((_BM, _N), lambda i: (i, 0)),
            pl.BlockSpec((_N, _D), lambda i: (0, 0)),
            pl.BlockSpec((_D, 2 * _D), lambda i: (0, 0)),
        ],
        out_specs=pl.BlockSpec((_BM, _D), lambda i: (i, 0)),
        out_shape=jax.ShapeDtypeStruct((_N, _D), jnp.float32),
        compiler_params=pltpu.CompilerParams(
            dimension_semantics=("parallel",)),
    )(input, adj, proj, W2)


# speedup vs baseline: 1.0199x; 1.0199x over previous
import jax
import jax.numpy as jnp
from jax.experimental import pallas as pl
from jax.experimental.pallas import tpu as pltpu

_N = 10000
_D = 128
_BM = 200


def _body(x_blk_ref, adj_ref, x_full_ref, W1_ref, W2_ref, out_ref, proj_ref):
    i = pl.program_id(0)

    @pl.when(i == 0)
    def _():
        proj_ref[...] = jax.lax.dot_general(
            x_full_ref[...], W1_ref[...],
            (((1,), (1,)), ((), ())),
            preferred_element_type=jnp.float32)

    nb = jax.lax.dot_general(
        adj_ref[...], proj_ref[...],
        (((1,), (0,)), ((), ())),
        preferred_element_type=jnp.float32)
    x = x_blk_ref[...]
    s = x + nb
    p = x * nb
    W2 = W2_ref[...]
    y = (jax.lax.dot_general(s, W2[:, :_D], (((1,), (1,)), ((), ())),
                             preferred_element_type=jnp.float32)
         + jax.lax.dot_general(p, W2[:, _D:], (((1,), (1,)), ((), ())),
                               preferred_element_type=jnp.float32))
    out_ref[...] = jnp.where(y >= 0.0, y, 0.01 * y)


def kernel(input, adj, W1, W2):
    return pl.pallas_call(
        _body,
        grid=(_N // _BM,),
        in_specs=[
            pl.BlockSpec((_BM, _D), lambda i: (i, 0)),
            pl.BlockSpec((_BM, _N), lambda i: (i, 0)),
            pl.BlockSpec((_N, _D), lambda i: (0, 0)),
            pl.BlockSpec((_D, _D), lambda i: (0, 0)),
            pl.BlockSpec((_D, 2 * _D), lambda i: (0, 0)),
        ],
        out_specs=pl.BlockSpec((_BM, _D), lambda i: (i, 0)),
        out_shape=jax.ShapeDtypeStruct((_N, _D), jnp.float32),
        scratch_shapes=[pltpu.VMEM((_N, _D), jnp.float32)],
    )(input, adj, input, W1, W2)


# speedup vs baseline: 1.0702x; 1.0494x over previous
import jax
import jax.numpy as jnp
from jax.experimental import pallas as pl
from jax.experimental.pallas import tpu as pltpu

_N = 10000
_D = 128
_BM = 400


def _body(adj_ref, x_full_ref, W1_ref, W2_ref, out_ref, proj_ref):
    i = pl.program_id(0)

    @pl.when(i == 0)
    def _():
        proj_ref[...] = jax.lax.dot_general(
            x_full_ref[...], W1_ref[...],
            (((1,), (1,)), ((), ())),
            preferred_element_type=jnp.float32)

    nb = jax.lax.dot_general(
        adj_ref[...], proj_ref[...],
        (((1,), (0,)), ((), ())),
        preferred_element_type=jnp.float32)
    x = x_full_ref[pl.ds(i * _BM, _BM), :]
    s = x + nb
    p = x * nb
    W2 = W2_ref[...]
    y = (jax.lax.dot_general(s, W2[:, :_D], (((1,), (1,)), ((), ())),
                             preferred_element_type=jnp.float32)
         + jax.lax.dot_general(p, W2[:, _D:], (((1,), (1,)), ((), ())),
                               preferred_element_type=jnp.float32))
    out_ref[...] = jnp.where(y >= 0.0, y, 0.01 * y)


def kernel(input, adj, W1, W2):
    return pl.pallas_call(
        _body,
        grid=(_N // _BM,),
        in_specs=[
            pl.BlockSpec((_BM, _N), lambda i: (i, 0)),
            pl.BlockSpec((_N, _D), lambda i: (0, 0)),
            pl.BlockSpec((_D, _D), lambda i: (0, 0)),
            pl.BlockSpec((_D, 2 * _D), lambda i: (0, 0)),
        ],
        out_specs=pl.BlockSpec((_BM, _D), lambda i: (i, 0)),
        out_shape=jax.ShapeDtypeStruct((_N, _D), jnp.float32),
        scratch_shapes=[pltpu.VMEM((_N, _D), jnp.float32)],
    )(adj, input, W1, W2)
